# baseline (device time: 30831 ns/iter reference)
import jax
import jax.numpy as jnp
from jax import lax
from jax.experimental import pallas as pl
from jax.experimental.pallas import tpu as pltpu

K = 16
KEY_MIN = jnp.iinfo(jnp.int32).min

OFFSETS = [
    (dx, dy, dz)
    for dx in (0, 1)
    for dy in (0, 1)
    for dz in (0, 1)
    if (dx, dy, dz) != (0, 0, 0)
]


def _encode(v, iota, n, idx_bits):
    i = lax.bitcast_convert_type(v, jnp.int32)
    key = jnp.where(i < 0, i ^ jnp.int32(0x7FFFFFFF), i)
    return (key & jnp.int32(~((1 << idx_bits) - 1))) | ((n - 1) - iota)


def _decode(key, idx_bits):
    k0 = key & jnp.int32(~((1 << idx_bits) - 1))
    i = jnp.where(k0 < 0, k0 ^ jnp.int32(0x7FFFFFFF), k0)
    return lax.bitcast_convert_type(i, jnp.float32)


def _bitonic_sort16(s, iota_k, descending):
    for d in (8, 4, 2, 1):
        low = (iota_k & d) == 0
        p = jnp.where(low, pltpu.roll(s, K - d, 1), pltpu.roll(s, d, 1))
        big, small = jnp.maximum(s, p), jnp.minimum(s, p)
        s = jnp.where(low, big, small) if descending else jnp.where(low, small, big)
    return s


def kernel(x):
    m, n = x.shape
    nq = n // 4
    qbits = (nq - 1).bit_length()

    def body(x_ref, out_ref, xq_ref, sbuf, rbuf, copy_sem, send_sems, recv_sems):
        my_x = lax.axis_index("x")
        my_y = lax.axis_index("y")
        my_z = lax.axis_index("z")

        def peer(off):
            dx, dy, dz = off
            return (
                (1 - my_x) if dx else my_x,
                (1 - my_y) if dy else my_y,
                (1 - my_z) if dz else my_z,
            )

        q = 2 * my_x + my_z
        copy = pltpu.make_async_copy(x_ref.at[:, pl.ds(q * nq, nq)], xq_ref, copy_sem)
        copy.start()

        barrier_sem = pltpu.get_barrier_semaphore()
        for off in OFFSETS:
            pl.semaphore_signal(
                barrier_sem, inc=1, device_id=peer(off),
                device_id_type=pl.DeviceIdType.MESH,
            )
        copy.wait()

        iota = lax.broadcasted_iota(jnp.int32, (m, nq), 1)
        keys = _encode(xq_ref[:, :], iota, nq, qbits)
        iota_k = lax.broadcasted_iota(jnp.int32, (m, K), 1)
        acc_d = jnp.full((m, K), KEY_MIN, jnp.int32)
        acc_a = jnp.full((m, K), KEY_MIN, jnp.int32)
        for i in range(K):
            mk = jnp.max(keys, axis=1, keepdims=True)
            acc_d = jnp.where(iota_k == i, mk, acc_d)
            acc_a = jnp.where(iota_k == K - 1 - i, mk, acc_a)
            keys = jnp.where(keys == mk, KEY_MIN, keys)
        a_desc = _decode(acc_d, qbits)
        sbuf[:, :] = _decode(acc_a, qbits)

        pl.semaphore_wait(barrier_sem, len(OFFSETS))

        rdmas = []
        for s, off in enumerate(OFFSETS):
            r = pltpu.make_async_remote_copy(
                src_ref=sbuf,
                dst_ref=rbuf.at[s],
                send_sem=send_sems.at[s],
                recv_sem=recv_sems.at[s],
                device_id=peer(off),
                device_id_type=pl.DeviceIdType.MESH,
            )
            r.start()
            rdmas.append(r)

        for s, r in enumerate(rdmas):
            r.wait_recv()
            a_desc = _bitonic_sort16(
                jnp.maximum(a_desc, rbuf[s]), iota_k, descending=True
            )
        out_ref[:, :] = a_desc

        for r in rdmas:
            r.wait_send()

    return pl.pallas_call(
        body,
        out_shape=jax.ShapeDtypeStruct((m, K), jnp.float32),
        in_specs=[pl.BlockSpec(memory_space=pl.MemorySpace.ANY)],
        out_specs=pl.BlockSpec(memory_space=pltpu.VMEM),
        scratch_shapes=[
            pltpu.VMEM((m, nq), jnp.float32),
            pltpu.VMEM((m, K), jnp.float32),
            pltpu.VMEM((7, m, K), jnp.float32),
            pltpu.SemaphoreType.DMA,
            pltpu.SemaphoreType.DMA((7,)),
            pltpu.SemaphoreType.DMA((7,)),
        ],
        compiler_params=pltpu.CompilerParams(collective_id=0),
    )(x)
